# baseline (speedup 1.0000x reference)
import functools
import jax
import jax.numpy as jnp
from jax import lax
from jax.experimental import pallas as pl
from jax.experimental.pallas import tpu as pltpu
from jax.experimental.pallas import tpu_sc as plsc

_B = 4
_S = 4096
_D = 1024
_NW = 32
_SPW = _S // _NW
_R = 16
_NJ = _SPW // _R
_NG = _NJ * _B


def _sc_body(x_hbm, pos_hbm, out_hbm,
             xb0, xb1, xb2, ob0, ob1, pb0, pb1,
             sin0, sin1, sin2, sout0, sout1, sp0, sp1):
    xbufs = (xb0, xb1, xb2)
    obufs = (ob0, ob1)
    pbufs = (pb0, pb1)
    sins = (sin0, sin1, sin2)
    souts = (sout0, sout1)
    sps = (sp0, sp1)

    wid = lax.axis_index("s") * 2 + lax.axis_index("c")
    base_seq = wid * _SPW

    def rows(g):
        j, b = divmod(g, _B)
        return b, base_seq + j * _R

    in_d = [None] * _NG
    out_d = [None] * _NG
    p_d = [None] * _NJ

    def start_in(g):
        b, r = rows(g)
        in_d[g] = pltpu.async_copy(
            x_hbm.at[b, pl.ds(r, _R), :], xbufs[g % 3], sins[g % 3])

    def start_pos(j):
        p_d[j] = pltpu.async_copy(
            pos_hbm.at[pl.ds(base_seq + j * _R, _R), :],
            pbufs[j % 2], sps[j % 2])

    start_pos(0)
    start_in(0)
    start_in(1)

    for g in range(_NG):
        j, b = divmod(g, _B)
        in_d[g].wait()
        if b == 0:
            p_d[j].wait()
            if j + 1 < _NJ:
                start_pos(j + 1)
        if g + 2 < _NG:
            start_in(g + 2)
        if g - 2 >= 0:
            out_d[g - 2].wait()

        xb = xbufs[g % 3]
        ob = obufs[g % 2]
        pb = pbufs[j % 2]

        def row_body(row, _):
            @plsc.parallel_loop(0, _D, step=16, unroll=32)
            def _add(i):
                sl = pl.ds(i, 16)
                ob[row, sl] = xb[row, sl] + pb[row, sl]
            return 0

        lax.fori_loop(0, _R, row_body, 0)

        b2, r2 = rows(g)
        out_d[g] = pltpu.async_copy(
            ob, out_hbm.at[b2, pl.ds(r2, _R), :], souts[g % 2])

    out_d[_NG - 2].wait()
    out_d[_NG - 1].wait()


def kernel(x, pos_table):
    batch, seq_len, d_model = x.shape

    mesh = plsc.VectorSubcoreMesh(core_axis_name="c", subcore_axis_name="s")
    k = functools.partial(
        pl.kernel,
        mesh=mesh,
        out_type=jax.ShapeDtypeStruct((batch, seq_len, d_model), x.dtype),
        scratch_types=[
            pltpu.VMEM((_R, _D), jnp.float32),
            pltpu.VMEM((_R, _D), jnp.float32),
            pltpu.VMEM((_R, _D), jnp.float32),
            pltpu.VMEM((_R, _D), jnp.float32),
            pltpu.VMEM((_R, _D), jnp.float32),
            pltpu.VMEM((_R, _D), jnp.float32),
            pltpu.VMEM((_R, _D), jnp.float32),
            pltpu.SemaphoreType.DMA,
            pltpu.SemaphoreType.DMA,
            pltpu.SemaphoreType.DMA,
            pltpu.SemaphoreType.DMA,
            pltpu.SemaphoreType.DMA,
            pltpu.SemaphoreType.DMA,
            pltpu.SemaphoreType.DMA,
        ],
    )(_sc_body)
    return k(x, pos_table)


# speedup vs baseline: 1.0103x; 1.0103x over previous
import functools
import jax
import jax.numpy as jnp
from jax import lax
from jax.experimental import pallas as pl
from jax.experimental.pallas import tpu as pltpu
from jax.experimental.pallas import tpu_sc as plsc

_B = 4
_S = 4096
_D = 1024
_NW = 32
_SPW = _S // _NW
_R = 16
_NJ = _SPW // _R
_NG = _NJ * _B


def _sc_body(x_hbm, pos_hbm, out_hbm,
             xb0, xb1, xb2, ob0, ob1, pb0, pb1,
             sin0, sin1, sin2, sout0, sout1, sp0, sp1):
    xbufs = (xb0, xb1, xb2)
    obufs = (ob0, ob1)
    pbufs = (pb0, pb1)
    sins = (sin0, sin1, sin2)
    souts = (sout0, sout1)
    sps = (sp0, sp1)

    wid = lax.axis_index("s") * 2 + lax.axis_index("c")
    base_seq = wid * _SPW

    def rows(g):
        j, b = divmod(g, _B)
        return b, base_seq + j * _R

    in_d = [None] * _NG
    out_d = [None] * _NG
    p_d = [None] * _NJ

    def start_in(g):
        b, r = rows(g)
        in_d[g] = pltpu.async_copy(
            x_hbm.at[b, pl.ds(r, _R), :], xbufs[g % 3], sins[g % 3])

    def start_pos(j):
        p_d[j] = pltpu.async_copy(
            pos_hbm.at[pl.ds(base_seq + j * _R, _R), :],
            pbufs[j % 2], sps[j % 2])

    start_pos(0)
    start_in(0)
    start_in(1)

    for g in range(_NG):
        j, b = divmod(g, _B)
        in_d[g].wait()
        if b == 0:
            p_d[j].wait()
            if j + 1 < _NJ:
                start_pos(j + 1)
        if g + 2 < _NG:
            start_in(g + 2)
        if g - 2 >= 0:
            out_d[g - 2].wait()

        xb = xbufs[g % 3]
        ob = obufs[g % 2]
        pb = pbufs[j % 2]

        def row_body(row, _):
            @plsc.parallel_loop(0, _D, step=16, unroll=16)
            def _add(i):
                sl = pl.ds(i, 16)
                ob[row, sl] = xb[row, sl] + pb[row, sl]
            return 0

        lax.fori_loop(0, _R, row_body, 0)

        b2, r2 = rows(g)
        out_d[g] = pltpu.async_copy(
            ob, out_hbm.at[b2, pl.ds(r2, _R), :], souts[g % 2])

    out_d[_NG - 2].wait()
    out_d[_NG - 1].wait()


def kernel(x, pos_table):
    batch, seq_len, d_model = x.shape

    mesh = plsc.VectorSubcoreMesh(core_axis_name="c", subcore_axis_name="s")
    k = functools.partial(
        pl.kernel,
        mesh=mesh,
        out_type=jax.ShapeDtypeStruct((batch, seq_len, d_model), x.dtype),
        scratch_types=[
            pltpu.VMEM((_R, _D), jnp.float32),
            pltpu.VMEM((_R, _D), jnp.float32),
            pltpu.VMEM((_R, _D), jnp.float32),
            pltpu.VMEM((_R, _D), jnp.float32),
            pltpu.VMEM((_R, _D), jnp.float32),
            pltpu.VMEM((_R, _D), jnp.float32),
            pltpu.VMEM((_R, _D), jnp.float32),
            pltpu.SemaphoreType.DMA,
            pltpu.SemaphoreType.DMA,
            pltpu.SemaphoreType.DMA,
            pltpu.SemaphoreType.DMA,
            pltpu.SemaphoreType.DMA,
            pltpu.SemaphoreType.DMA,
            pltpu.SemaphoreType.DMA,
        ],
    )(_sc_body)
    return k(x, pos_table)


# speedup vs baseline: 1.0382x; 1.0276x over previous
import functools
import jax
import jax.numpy as jnp
from jax import lax
from jax.experimental import pallas as pl
from jax.experimental.pallas import tpu as pltpu
from jax.experimental.pallas import tpu_sc as plsc

_B = 4
_S = 4096
_D = 1024
_NW = 32
_SPW = _S // _NW
_R = 16
_NJ = _SPW // _R
_NG = _NJ * _B


def _sc_body(x_hbm, pos_hbm, out_hbm,
             xb0, xb1, xb2, ob0, ob1, pb0, pb1,
             sin0, sin1, sin2, sout0, sout1, sp0, sp1):
    xbufs = (xb0, xb1, xb2)
    obufs = (ob0, ob1)
    pbufs = (pb0, pb1)
    sins = (sin0, sin1, sin2)
    souts = (sout0, sout1)
    sps = (sp0, sp1)

    wid = lax.axis_index("s") * 2 + lax.axis_index("c")
    base_seq = wid * _SPW

    def rows(g):
        j, b = divmod(g, _B)
        return b, base_seq + j * _R

    in_d = [None] * _NG
    out_d = [None] * _NG
    p_d = [None] * _NJ

    def start_in(g):
        b, r = rows(g)
        in_d[g] = pltpu.async_copy(
            x_hbm.at[b, pl.ds(r, _R), :], xbufs[g % 3], sins[g % 3])

    def start_pos(j):
        p_d[j] = pltpu.async_copy(
            pos_hbm.at[pl.ds(base_seq + j * _R, _R), :],
            pbufs[j % 2], sps[j % 2])

    start_pos(0)
    start_in(0)
    start_in(1)

    for g in range(_NG):
        j, b = divmod(g, _B)
        in_d[g].wait()
        if b == 0:
            p_d[j].wait()
            if j + 1 < _NJ:
                start_pos(j + 1)
        if g + 2 < _NG:
            start_in(g + 2)
        if g - 2 >= 0:
            out_d[g - 2].wait()

        xb = xbufs[g % 3]
        ob = obufs[g % 2]
        pb = pbufs[j % 2]

        def row_body(row, _):
            @plsc.parallel_loop(0, _D, step=16, unroll=8)
            def _add(i):
                sl = pl.ds(i, 16)
                ob[row, sl] = xb[row, sl] + pb[row, sl]
            return 0

        lax.fori_loop(0, _R, row_body, 0)

        b2, r2 = rows(g)
        out_d[g] = pltpu.async_copy(
            ob, out_hbm.at[b2, pl.ds(r2, _R), :], souts[g % 2])

    out_d[_NG - 2].wait()
    out_d[_NG - 1].wait()


def kernel(x, pos_table):
    batch, seq_len, d_model = x.shape

    mesh = plsc.VectorSubcoreMesh(core_axis_name="c", subcore_axis_name="s")
    k = functools.partial(
        pl.kernel,
        mesh=mesh,
        out_type=jax.ShapeDtypeStruct((batch, seq_len, d_model), x.dtype),
        scratch_types=[
            pltpu.VMEM((_R, _D), jnp.float32),
            pltpu.VMEM((_R, _D), jnp.float32),
            pltpu.VMEM((_R, _D), jnp.float32),
            pltpu.VMEM((_R, _D), jnp.float32),
            pltpu.VMEM((_R, _D), jnp.float32),
            pltpu.VMEM((_R, _D), jnp.float32),
            pltpu.VMEM((_R, _D), jnp.float32),
            pltpu.SemaphoreType.DMA,
            pltpu.SemaphoreType.DMA,
            pltpu.SemaphoreType.DMA,
            pltpu.SemaphoreType.DMA,
            pltpu.SemaphoreType.DMA,
            pltpu.SemaphoreType.DMA,
            pltpu.SemaphoreType.DMA,
        ],
    )(_sc_body)
    return k(x, pos_table)


# speedup vs baseline: 1.0492x; 1.0106x over previous
import functools
import jax
import jax.numpy as jnp
from jax import lax
from jax.experimental import pallas as pl
from jax.experimental.pallas import tpu as pltpu
from jax.experimental.pallas import tpu_sc as plsc

_B = 4
_S = 4096
_D = 1024
_NW = 32
_SPW = _S // _NW
_R = 16
_NJ = _SPW // _R
_NG = _NJ * _B


def _sc_body(x_hbm, pos_hbm, out_hbm,
             xb0, xb1, xb2, ob0, ob1, pb0, pb1,
             sin0, sin1, sin2, sout0, sout1, sp0, sp1):
    xbufs = (xb0, xb1, xb2)
    obufs = (ob0, ob1)
    pbufs = (pb0, pb1)
    sins = (sin0, sin1, sin2)
    souts = (sout0, sout1)
    sps = (sp0, sp1)

    wid = lax.axis_index("s") * 2 + lax.axis_index("c")
    base_seq = wid * _SPW

    def rows(g):
        j, b = divmod(g, _B)
        return b, base_seq + j * _R

    in_d = [None] * _NG
    out_d = [None] * _NG
    p_d = [None] * _NJ

    def start_in(g):
        b, r = rows(g)
        in_d[g] = pltpu.async_copy(
            x_hbm.at[b, pl.ds(r, _R), :], xbufs[g % 3], sins[g % 3])

    def start_pos(j):
        p_d[j] = pltpu.async_copy(
            pos_hbm.at[pl.ds(base_seq + j * _R, _R), :],
            pbufs[j % 2], sps[j % 2])

    start_pos(0)
    start_in(0)
    start_in(1)

    for g in range(_NG):
        j, b = divmod(g, _B)
        in_d[g].wait()
        if b == 0:
            p_d[j].wait()
            if j + 1 < _NJ:
                start_pos(j + 1)
        if g + 2 < _NG:
            start_in(g + 2)
        if g - 2 >= 0:
            out_d[g - 2].wait()

        xb = xbufs[g % 3]
        ob = obufs[g % 2]
        pb = pbufs[j % 2]

        def row_body(row, _):
            @plsc.parallel_loop(0, _D, step=16, unroll=4)
            def _add(i):
                sl = pl.ds(i, 16)
                ob[row, sl] = xb[row, sl] + pb[row, sl]
            return 0

        lax.fori_loop(0, _R, row_body, 0)

        b2, r2 = rows(g)
        out_d[g] = pltpu.async_copy(
            ob, out_hbm.at[b2, pl.ds(r2, _R), :], souts[g % 2])

    out_d[_NG - 2].wait()
    out_d[_NG - 1].wait()


def kernel(x, pos_table):
    batch, seq_len, d_model = x.shape

    mesh = plsc.VectorSubcoreMesh(core_axis_name="c", subcore_axis_name="s")
    k = functools.partial(
        pl.kernel,
        mesh=mesh,
        out_type=jax.ShapeDtypeStruct((batch, seq_len, d_model), x.dtype),
        scratch_types=[
            pltpu.VMEM((_R, _D), jnp.float32),
            pltpu.VMEM((_R, _D), jnp.float32),
            pltpu.VMEM((_R, _D), jnp.float32),
            pltpu.VMEM((_R, _D), jnp.float32),
            pltpu.VMEM((_R, _D), jnp.float32),
            pltpu.VMEM((_R, _D), jnp.float32),
            pltpu.VMEM((_R, _D), jnp.float32),
            pltpu.SemaphoreType.DMA,
            pltpu.SemaphoreType.DMA,
            pltpu.SemaphoreType.DMA,
            pltpu.SemaphoreType.DMA,
            pltpu.SemaphoreType.DMA,
            pltpu.SemaphoreType.DMA,
            pltpu.SemaphoreType.DMA,
        ],
    )(_sc_body)
    return k(x, pos_table)
